# baseline (speedup 1.0000x reference)
import functools

import jax
import jax.numpy as jnp
from jax import lax
from jax.experimental import pallas as pl
from jax.experimental.pallas import tpu as pltpu
from jax.experimental.pallas import tpu_sc as plsc

N = 10000
NDEG = 10240
EROWS = 2500
RPT = 79
NPT = N // 16
D1 = 16
D2 = 40
RBLK = 2000

_MESH = dict(core_axis_name="c", subcore_axis_name="s")
_SC_PARAMS = pltpu.CompilerParams(use_tc_tiling_on_sc=False)


def _tile_rows(wid):
  rb = wid * 78 + jnp.maximum(wid - 28, 0)
  nblk = 78 + (wid >= 28).astype(jnp.int32)
  return rb, nblk


def _make_deg():
  mesh = plsc.VectorSubcoreMesh(**_MESH)

  @functools.partial(
      pl.kernel,
      out_type=jax.ShapeDtypeStruct((2, NDEG), jnp.float32),
      mesh=mesh,
      compiler_params=_SC_PARAMS,
      scratch_types=[
          pltpu.VMEM((RPT, 128), jnp.int32),
          pltpu.VMEM((128,), jnp.float32),
          pltpu.VMEM((NDEG // 16,), jnp.float32),
          pltpu.VMEM_SHARED((NDEG,), jnp.float32),
          pltpu.SemaphoreType.DMA,
      ],
  )
  def deg_kernel(dst_hbm, out_hbm, idx_v, ones_v, zero_v, acc, dsem):
    c = lax.axis_index("c")
    s = lax.axis_index("s")
    wid = c * 16 + s
    rb, nblk = _tile_rows(wid)
    npt = NDEG // 16
    one16 = jnp.full((16,), 1.0, jnp.float32)
    zero16 = jnp.zeros((16,), jnp.float32)
    for i in range(8):
      ones_v[pl.ds(i * 16, 16)] = one16

    def zbody(i, _):
      zero_v[pl.ds(i * 16, 16)] = zero16
      return 0

    lax.fori_loop(0, npt // 16, zbody, 0)
    pltpu.sync_copy(zero_v, acc.at[pl.ds(s * npt, npt)])
    pltpu.sync_copy(dst_hbm.at[pl.ds(rb, RPT)], idx_v)
    plsc.subcore_barrier()

    def body(j, _):
      pltpu.async_copy(ones_v, acc.at[idx_v.at[j]], dsem, add=True)
      return 0

    lax.fori_loop(0, nblk, body, 0)

    def drain(j, _):
      pltpu.make_async_copy(ones_v, acc.at[idx_v.at[j]], dsem).wait()
      return 0

    lax.fori_loop(0, nblk, drain, 0)
    plsc.subcore_barrier()
    pltpu.sync_copy(acc.at[pl.ds(s * npt, npt)],
                    out_hbm.at[c, pl.ds(s * npt, npt)])

  return deg_kernel


def _make_pass(d):
  mesh = plsc.VectorSubcoreMesh(**_MESH)

  @functools.partial(
      pl.kernel,
      out_type=jax.ShapeDtypeStruct((2, N, d), jnp.float32),
      mesh=mesh,
      compiler_params=_SC_PARAMS,
      scratch_types=[
          pltpu.VMEM((RPT, 128), jnp.int32),
          pltpu.VMEM((RPT, 128), jnp.int32),
          [pltpu.VMEM((128, d), jnp.float32) for _ in range(4)],
          pltpu.VMEM_SHARED((N, d), jnp.float32),
          pltpu.VMEM_SHARED((N, d), jnp.float32),
          [pltpu.SemaphoreType.DMA for _ in range(4)],
          [pltpu.SemaphoreType.DMA for _ in range(4)],
      ],
  )
  def pass_kernel(hs_hbm, src_hbm, dst_hbm, zz_hbm, out_hbm,
                  sidx, didx, rows, table, acc, gsem, ssem):
    c = lax.axis_index("c")
    s = lax.axis_index("s")
    wid = c * 16 + s
    rb, nblk = _tile_rows(wid)
    pltpu.sync_copy(hs_hbm.at[pl.ds(s * NPT, NPT)],
                    table.at[pl.ds(s * NPT, NPT)])
    pltpu.sync_copy(zz_hbm, acc.at[pl.ds(s * NPT, NPT)])
    pltpu.sync_copy(src_hbm.at[pl.ds(rb, RPT)], sidx)
    pltpu.sync_copy(dst_hbm.at[pl.ds(rb, RPT)], didx)
    plsc.subcore_barrier()

    def gather(b, u):
      pltpu.async_copy(table.at[sidx.at[b]], rows[u], gsem[u])

    def wait_gather(b, u):
      pltpu.make_async_copy(table.at[sidx.at[b]], rows[u], gsem[u]).wait()

    def scatter(b, u):
      pltpu.async_copy(rows[u], acc.at[didx.at[b]], ssem[u], add=True)

    def wait_scatter(b, u):
      pltpu.make_async_copy(rows[u], acc.at[didx.at[b]], ssem[u]).wait()

    for u in range(4):
      gather(u, u)

    def body(i, _):
      for u in range(4):
        b = 4 * i + u
        wait_gather(b, u)
        scatter(b, u)
      for u in range(4):
        b = 4 * i + u
        bn = b + 4
        wait_scatter(b, u)

        @pl.when(bn < nblk)
        def _():
          gather(bn, u)

      return 0

    lax.fori_loop(0, 19, body, 0)

    for u in range(4):
      b = 76 + u
      if u < 2:
        wait_gather(b, u)
        scatter(b, u)
      else:

        @pl.when(b < nblk)
        def _():
          wait_gather(b, u)
          scatter(b, u)

    for u in range(4):
      b = 76 + u
      if u < 2:
        wait_scatter(b, u)
      else:

        @pl.when(b < nblk)
        def _():
          wait_scatter(b, u)

    plsc.subcore_barrier()
    pltpu.sync_copy(acc.at[pl.ds(s * NPT, NPT)],
                    out_hbm.at[c, pl.ds(s * NPT, NPT)])

  return pass_kernel


_deg_call = _make_deg()
_pass16 = _make_pass(D1)
_pass40 = _make_pass(D2)


def _stage_a_body(x_ref, w_ref, d_ref, o_ref):
  o_ref[:, :] = jnp.dot(x_ref[:, :], w_ref[:, :],
                        preferred_element_type=jnp.float32) * d_ref[:, :]


def _stage_a(x, w1, dinv_col):
  return pl.pallas_call(
      _stage_a_body,
      out_shape=jax.ShapeDtypeStruct((N, D1), jnp.float32),
  )(x, w1, dinv_col)


def _stage_b_body(p_ref, h_ref, d_ref, b1_ref, w2_ref, o_ref):
  dcol = d_ref[:, :]
  t = (p_ref[0] + p_ref[1] + h_ref[:, :]) * dcol + b1_ref[:, :]
  t = jnp.maximum(t, 0.0)
  o_ref[:, :] = jnp.dot(t, w2_ref[:, :],
                        preferred_element_type=jnp.float32) * dcol


def _stage_b(p1, hs1, dinv_col, b1r, w2):
  return pl.pallas_call(
      _stage_b_body,
      out_shape=jax.ShapeDtypeStruct((N, D2), jnp.float32),
  )(p1, hs1, dinv_col, b1r, w2)


def _stage_c_body(p_ref, h_ref, d_ref, b2_ref, o_ref):
  logits = ((p_ref[0] + p_ref[1] + h_ref[:, :]) * d_ref[:, :]
            + b2_ref[:, :])
  m = jnp.max(logits, axis=1, keepdims=True)
  e = jnp.exp(logits - m)
  ssum = jnp.sum(e, axis=1, keepdims=True)
  o_ref[:, :] = logits - m - jnp.log(ssum)


def _stage_c(p2, hs2, dinv_col, b2r):
  return pl.pallas_call(
      _stage_c_body,
      out_shape=jax.ShapeDtypeStruct((N, D2), jnp.float32),
  )(p2, hs2, dinv_col, b2r)


def kernel(x, edge_index, W1, b1, W2, b2):
  ei = edge_index.astype(jnp.int32).reshape(2, EROWS, 128)
  src2d = ei[0]
  dst2d = ei[1]

  deg_p = _deg_call(dst2d)
  deg = deg_p[0, :N] + deg_p[1, :N] + 1.0
  dinv_col = lax.rsqrt(deg).reshape(N, 1)
  zz16 = jnp.zeros((NPT, D1), jnp.float32)
  zz40 = jnp.zeros((NPT, D2), jnp.float32)

  hs1 = _stage_a(x, W1, dinv_col)
  p1 = _pass16(hs1, src2d, dst2d, zz16)

  b1r = b1.reshape(1, D1)
  b2r = b2.reshape(1, D2)

  hs2 = _stage_b(p1, hs1, dinv_col, b1r, W2)
  p2 = _pass40(hs2, src2d, dst2d, zz40)
  return _stage_c(p2, hs2, dinv_col, b2r)


# speedup vs baseline: 1.1738x; 1.1738x over previous
import functools

import jax
import jax.numpy as jnp
from jax import lax
from jax.experimental import pallas as pl
from jax.experimental.pallas import tpu as pltpu
from jax.experimental.pallas import tpu_sc as plsc

N = 10000
NDEG = 10240
EROWS = 2500
RPT = 79
NPT = N // 16
D1 = 16
D2 = 40
RBLK = 2000

_MESH = dict(core_axis_name="c", subcore_axis_name="s")
_SC_PARAMS = pltpu.CompilerParams(use_tc_tiling_on_sc=False)


def _tile_rows(wid):
  rb = wid * 78 + jnp.maximum(wid - 28, 0)
  nblk = 78 + (wid >= 28).astype(jnp.int32)
  return rb, nblk


def _make_deg():
  mesh = plsc.VectorSubcoreMesh(**_MESH)

  @functools.partial(
      pl.kernel,
      out_type=jax.ShapeDtypeStruct((2, NDEG), jnp.float32),
      mesh=mesh,
      compiler_params=_SC_PARAMS,
      scratch_types=[
          pltpu.VMEM((RPT, 128), jnp.int32),
          pltpu.VMEM((128,), jnp.float32),
          pltpu.VMEM((NDEG // 16,), jnp.float32),
          pltpu.VMEM_SHARED((NDEG,), jnp.float32),
      ],
  )
  def deg_kernel(dst_hbm, out_hbm, idx_v, ones_v, zero_v, acc):
    c = lax.axis_index("c")
    s = lax.axis_index("s")
    wid = c * 16 + s
    rb, nblk = _tile_rows(wid)
    npt = NDEG // 16
    one16 = jnp.full((16,), 1.0, jnp.float32)
    zero16 = jnp.zeros((16,), jnp.float32)
    for i in range(8):
      ones_v[pl.ds(i * 16, 16)] = one16

    def zbody(i, _):
      zero_v[pl.ds(i * 16, 16)] = zero16
      return 0

    lax.fori_loop(0, npt // 16, zbody, 0)
    pltpu.sync_copy(zero_v, acc.at[pl.ds(s * npt, npt)])
    pltpu.sync_copy(dst_hbm.at[pl.ds(rb, RPT)], idx_v)
    plsc.subcore_barrier()

    def body(j, _):
      pltpu.sync_copy(ones_v, acc.at[idx_v.at[j]], add=True)
      return 0

    lax.fori_loop(0, nblk, body, 0)
    plsc.subcore_barrier()
    pltpu.sync_copy(acc.at[pl.ds(s * npt, npt)],
                    out_hbm.at[c, pl.ds(s * npt, npt)])

  return deg_kernel


def _make_pass(d):
  mesh = plsc.VectorSubcoreMesh(**_MESH)

  @functools.partial(
      pl.kernel,
      out_type=jax.ShapeDtypeStruct((2, N, 128), jnp.float32),
      mesh=mesh,
      compiler_params=_SC_PARAMS,
      scratch_types=[
          pltpu.VMEM((RPT, 128), jnp.int32),
          pltpu.VMEM((RPT, 128), jnp.int32),
          pltpu.VMEM((128, d), jnp.float32),
          pltpu.VMEM((128, d), jnp.float32),
          pltpu.VMEM_SHARED((N, d), jnp.float32),
          pltpu.VMEM_SHARED((N, d), jnp.float32),
          pltpu.SemaphoreType.DMA,
          pltpu.SemaphoreType.DMA,
      ],
  )
  def pass_kernel(hs_hbm, src_hbm, dst_hbm, zz_hbm, out_hbm,
                  sidx, didx, rows0, rows1, table, acc, sem0, sem1):
    c = lax.axis_index("c")
    s = lax.axis_index("s")
    wid = c * 16 + s
    rb, nblk = _tile_rows(wid)
    pltpu.sync_copy(hs_hbm.at[pl.ds(s * NPT, NPT), pl.ds(0, d)],
                    table.at[pl.ds(s * NPT, NPT)])
    pltpu.sync_copy(zz_hbm, acc.at[pl.ds(s * NPT, NPT)])
    pltpu.sync_copy(src_hbm.at[pl.ds(rb, RPT)], sidx)
    pltpu.sync_copy(dst_hbm.at[pl.ds(rb, RPT)], didx)
    plsc.subcore_barrier()

    pltpu.async_copy(table.at[sidx.at[0]], rows0, sem0)

    def body(i, _):
      b0 = 2 * i
      b1 = 2 * i + 1
      pltpu.async_copy(table.at[sidx.at[b1]], rows1, sem1)
      pltpu.make_async_copy(table.at[sidx.at[b0]], rows0, sem0).wait()
      pltpu.sync_copy(rows0, acc.at[didx.at[b0]], add=True)

      @pl.when(b0 + 2 < nblk)
      def _():
        pltpu.async_copy(table.at[sidx.at[b0 + 2]], rows0, sem0)

      pltpu.make_async_copy(table.at[sidx.at[b1]], rows1, sem1).wait()
      pltpu.sync_copy(rows1, acc.at[didx.at[b1]], add=True)
      return 0

    lax.fori_loop(0, 39, body, 0)

    @pl.when(nblk == RPT)
    def _():
      pltpu.make_async_copy(table.at[sidx.at[RPT - 1]], rows0, sem0).wait()
      pltpu.sync_copy(rows0, acc.at[didx.at[RPT - 1]], add=True)

    plsc.subcore_barrier()
    pltpu.sync_copy(acc.at[pl.ds(s * NPT, NPT)],
                    out_hbm.at[c, pl.ds(s * NPT, NPT), pl.ds(0, d)])

  return pass_kernel


_deg_call = _make_deg()
_pass16 = _make_pass(D1)
_pass40 = _make_pass(D2)


def _stage_a_body(x_ref, w_ref, d_ref, o_ref):
  o_ref[:, :D1] = jnp.dot(x_ref[:, :], w_ref[:, :],
                          preferred_element_type=jnp.float32) * d_ref[:, :]


def _stage_a(x, w1, dinv_col):
  return pl.pallas_call(
      _stage_a_body,
      out_shape=jax.ShapeDtypeStruct((N, 128), jnp.float32),
  )(x, w1, dinv_col)


def _stage_b_body(p_ref, h_ref, d_ref, b1_ref, w2_ref, o_ref):
  dcol = d_ref[:, :]
  t = ((p_ref[0, :, :D1] + p_ref[1, :, :D1] + h_ref[:, :D1]) * dcol
       + b1_ref[:, :])
  t = jnp.maximum(t, 0.0)
  o_ref[:, :D2] = jnp.dot(t, w2_ref[:, :],
                          preferred_element_type=jnp.float32) * dcol


def _stage_b(p1, hs1, dinv_col, b1r, w2):
  return pl.pallas_call(
      _stage_b_body,
      out_shape=jax.ShapeDtypeStruct((N, 128), jnp.float32),
  )(p1, hs1, dinv_col, b1r, w2)


def _stage_c_body(p_ref, h_ref, d_ref, b2_ref, o_ref):
  logits = ((p_ref[0, :, :D2] + p_ref[1, :, :D2] + h_ref[:, :D2])
            * d_ref[:, :] + b2_ref[:, :])
  m = jnp.max(logits, axis=1, keepdims=True)
  e = jnp.exp(logits - m)
  ssum = jnp.sum(e, axis=1, keepdims=True)
  o_ref[:, :] = logits - m - jnp.log(ssum)


def _stage_c(p2, hs2, dinv_col, b2r):
  return pl.pallas_call(
      _stage_c_body,
      out_shape=jax.ShapeDtypeStruct((N, D2), jnp.float32),
  )(p2, hs2, dinv_col, b2r)


def kernel(x, edge_index, W1, b1, W2, b2):
  ei = edge_index.astype(jnp.int32).reshape(2, EROWS, 128)
  src2d = ei[0]
  dst2d = ei[1]

  deg_p = _deg_call(dst2d)
  deg = deg_p[0, :N] + deg_p[1, :N] + 1.0
  dinv_col = lax.rsqrt(deg).reshape(N, 1)
  zz16 = jnp.zeros((NPT, D1), jnp.float32)
  zz40 = jnp.zeros((NPT, D2), jnp.float32)

  hs1 = _stage_a(x, W1, dinv_col)
  p1 = _pass16(hs1, src2d, dst2d, zz16)

  b1r = b1.reshape(1, D1)
  b2r = b2.reshape(1, D2)

  hs2 = _stage_b(p1, hs1, dinv_col, b1r, W2)
  p2 = _pass40(hs2, src2d, dst2d, zz40)
  return _stage_c(p2, hs2, dinv_col, b2r)


# speedup vs baseline: 1.1888x; 1.0128x over previous
import functools

import jax
import jax.numpy as jnp
from jax import lax
from jax.experimental import pallas as pl
from jax.experimental.pallas import tpu as pltpu
from jax.experimental.pallas import tpu_sc as plsc

N = 10000
NDEG = 10240
EROWS = 2500
RPT = 79
NPT = N // 16
D1 = 16
D2 = 40
RBLK = 2000

_MESH = dict(core_axis_name="c", subcore_axis_name="s")
_SC_PARAMS = pltpu.CompilerParams(use_tc_tiling_on_sc=False,
                                  needs_layout_passes=False)


def _tile_rows(wid):
  rb = wid * 78 + jnp.maximum(wid - 28, 0)
  nblk = 78 + (wid >= 28).astype(jnp.int32)
  return rb, nblk


def _make_deg():
  mesh = plsc.VectorSubcoreMesh(**_MESH)

  @functools.partial(
      pl.kernel,
      out_type=jax.ShapeDtypeStruct((2, NDEG), jnp.float32),
      mesh=mesh,
      compiler_params=_SC_PARAMS,
      scratch_types=[
          pltpu.VMEM((RPT, 128), jnp.int32),
          pltpu.VMEM((128,), jnp.float32),
          pltpu.VMEM((NDEG // 16,), jnp.float32),
          pltpu.VMEM_SHARED((NDEG,), jnp.float32),
          pltpu.SemaphoreType.DMA,
      ],
  )
  def deg_kernel(dst_hbm, out_hbm, idx_v, ones_v, zero_v, acc, dsem):
    c = lax.axis_index("c")
    s = lax.axis_index("s")
    wid = c * 16 + s
    rb, nblk = _tile_rows(wid)
    npt = NDEG // 16
    one16 = jnp.full((16,), 1.0, jnp.float32)
    zero16 = jnp.zeros((16,), jnp.float32)
    for i in range(8):
      ones_v[pl.ds(i * 16, 16)] = one16

    def zbody(i, _):
      zero_v[pl.ds(i * 16, 16)] = zero16
      return 0

    lax.fori_loop(0, npt // 16, zbody, 0)
    pltpu.sync_copy(zero_v, acc.at[pl.ds(s * npt, npt)])
    pltpu.sync_copy(dst_hbm.at[pl.ds(rb, RPT)], idx_v)
    plsc.subcore_barrier()

    def body(j, _):
      pltpu.async_copy(ones_v, acc.at[idx_v.at[j]], dsem, add=True)
      return 0

    lax.fori_loop(0, nblk, body, 0)

    def drain(j, _):
      pltpu.make_async_copy(ones_v, acc.at[idx_v.at[j]], dsem).wait()
      return 0

    lax.fori_loop(0, nblk, drain, 0)
    plsc.subcore_barrier()
    pltpu.sync_copy(acc.at[pl.ds(s * npt, npt)],
                    out_hbm.at[c, pl.ds(s * npt, npt)])

  return deg_kernel


def _make_pass(d, scaled=False):
  mesh = plsc.VectorSubcoreMesh(**_MESH)
  scratch = [
      pltpu.VMEM((RPT, 128), jnp.int32),
      pltpu.VMEM((RPT, 128), jnp.int32),
      pltpu.VMEM((128, d), jnp.float32),
      pltpu.VMEM((128, d), jnp.float32),
      pltpu.VMEM_SHARED((N, d), jnp.float32),
      pltpu.VMEM_SHARED((N, d), jnp.float32),
      pltpu.SemaphoreType.DMA,
      pltpu.SemaphoreType.DMA,
  ]
  if scaled:
    scratch += [
        pltpu.VMEM((NPT, d), jnp.float32),
        pltpu.VMEM((NPT + 7, ), jnp.float32),
    ]

  def pass_body(hs_hbm, dinv_hbm, src_hbm, dst_hbm, zz_hbm, out_hbm,
                sidx, didx, rows0, rows1, table, acc, sem0, sem1,
                tmp=None, dvec=None):
    c = lax.axis_index("c")
    s = lax.axis_index("s")
    wid = c * 16 + s
    rb, nblk = _tile_rows(wid)
    if not scaled:
      pltpu.sync_copy(hs_hbm.at[pl.ds(s * NPT, NPT), pl.ds(0, d)],
                      table.at[pl.ds(s * NPT, NPT)])
    else:
      pltpu.sync_copy(hs_hbm.at[pl.ds(s * NPT, NPT), pl.ds(0, d)], tmp)
      b8 = s * NPT // 8 * 8
      off = s * NPT - b8
      pltpu.sync_copy(dinv_hbm.at[pl.ds(b8, NPT + 7)], dvec)

      def scale_row(i, _):
        dv = plsc.load_gather(
            dvec, [jnp.zeros((16,), jnp.int32) + (off + i)])
        tmp[i] = tmp[i] * dv
        return 0

      lax.fori_loop(0, NPT, scale_row, 0)
      pltpu.sync_copy(tmp, table.at[pl.ds(s * NPT, NPT)])
    pltpu.sync_copy(zz_hbm, acc.at[pl.ds(s * NPT, NPT)])
    pltpu.sync_copy(src_hbm.at[pl.ds(rb, RPT)], sidx)
    pltpu.sync_copy(dst_hbm.at[pl.ds(rb, RPT)], didx)
    plsc.subcore_barrier()

    pltpu.async_copy(table.at[sidx.at[0]], rows0, sem0)

    def body(i, _):
      b0 = 2 * i
      b1 = 2 * i + 1
      pltpu.async_copy(table.at[sidx.at[b1]], rows1, sem1)
      pltpu.make_async_copy(table.at[sidx.at[b0]], rows0, sem0).wait()
      pltpu.sync_copy(rows0, acc.at[didx.at[b0]], add=True)

      @pl.when(b0 + 2 < nblk)
      def _():
        pltpu.async_copy(table.at[sidx.at[b0 + 2]], rows0, sem0)

      pltpu.make_async_copy(table.at[sidx.at[b1]], rows1, sem1).wait()
      pltpu.sync_copy(rows1, acc.at[didx.at[b1]], add=True)
      return 0

    lax.fori_loop(0, 39, body, 0)

    @pl.when(nblk == RPT)
    def _():
      pltpu.make_async_copy(table.at[sidx.at[RPT - 1]], rows0, sem0).wait()
      pltpu.sync_copy(rows0, acc.at[didx.at[RPT - 1]], add=True)

    plsc.subcore_barrier()
    pltpu.sync_copy(acc.at[pl.ds(s * NPT, NPT)],
                    out_hbm.at[c, pl.ds(s * NPT, NPT), pl.ds(0, d)])

  kern = functools.partial(
      pl.kernel,
      out_type=jax.ShapeDtypeStruct((2, N, 128), jnp.float32),
      mesh=mesh,
      compiler_params=_SC_PARAMS,
      scratch_types=scratch,
  )
  if scaled:
    return kern(pass_body)

  def body_unscaled(hs_hbm, src_hbm, dst_hbm, zz_hbm, out_hbm, *rest):
    pass_body(hs_hbm, None, src_hbm, dst_hbm, zz_hbm, out_hbm, *rest)

  return kern(body_unscaled)


_deg_call = _make_deg()
_pass16 = _make_pass(D1, scaled=True)
_pass40 = _make_pass(D2)


def _stage_a_body(x_ref, w_ref, o_ref):
  o_ref[:, :D1] = jnp.dot(x_ref[:, :], w_ref[:, :],
                          preferred_element_type=jnp.float32)


def _stage_a(x, w1):
  return pl.pallas_call(
      _stage_a_body,
      out_shape=jax.ShapeDtypeStruct((N, 128), jnp.float32),
  )(x, w1)


def _stage_b_body(p_ref, h_ref, d_ref, b1_ref, w2_ref, o_ref):
  dcol = d_ref[:, :]
  t = ((p_ref[0, :, :D1] + p_ref[1, :, :D1] + h_ref[:, :D1] * dcol) * dcol
       + b1_ref[:, :])
  t = jnp.maximum(t, 0.0)
  o_ref[:, :D2] = jnp.dot(t, w2_ref[:, :],
                          preferred_element_type=jnp.float32) * dcol


def _stage_b(p1, hs1, dinv_col, b1r, w2):
  return pl.pallas_call(
      _stage_b_body,
      out_shape=jax.ShapeDtypeStruct((N, 128), jnp.float32),
  )(p1, hs1, dinv_col, b1r, w2)


def _stage_c_body(p_ref, h_ref, d_ref, b2_ref, o_ref):
  logits = ((p_ref[0, :, :D2] + p_ref[1, :, :D2] + h_ref[:, :D2])
            * d_ref[:, :] + b2_ref[:, :])
  m = jnp.max(logits, axis=1, keepdims=True)
  e = jnp.exp(logits - m)
  ssum = jnp.sum(e, axis=1, keepdims=True)
  o_ref[:, :] = logits - m - jnp.log(ssum)


def _stage_c(p2, hs2, dinv_col, b2r):
  return pl.pallas_call(
      _stage_c_body,
      out_shape=jax.ShapeDtypeStruct((N, D2), jnp.float32),
  )(p2, hs2, dinv_col, b2r)


def kernel(x, edge_index, W1, b1, W2, b2):
  ei = edge_index.astype(jnp.int32).reshape(2, EROWS, 128)
  src2d = ei[0]
  dst2d = ei[1]

  deg_p = _deg_call(dst2d)
  deg = deg_p[0, :N] + deg_p[1, :N] + 1.0
  dinv = lax.rsqrt(deg)
  dinv_col = dinv.reshape(N, 1)
  zz16 = jnp.zeros((NPT, D1), jnp.float32)
  zz40 = jnp.zeros((NPT, D2), jnp.float32)

  h1 = _stage_a(x, W1)
  p1 = _pass16(h1, dinv, src2d, dst2d, zz16)

  b1r = b1.reshape(1, D1)
  b2r = b2.reshape(1, D2)

  hs2 = _stage_b(p1, h1, dinv_col, b1r, W2)
  p2 = _pass40(hs2, src2d, dst2d, zz40)
  return _stage_c(p2, hs2, dinv_col, b2r)


# speedup vs baseline: 1.1994x; 1.0090x over previous
import functools

import jax
import jax.numpy as jnp
from jax import lax
from jax.experimental import pallas as pl
from jax.experimental.pallas import tpu as pltpu
from jax.experimental.pallas import tpu_sc as plsc

N = 10000
NDEG = 10240
EROWS = 2500
RPT = 79
NPT = N // 16
D1 = 16
D2 = 40
RBLK = 2000

_MESH = dict(core_axis_name="c", subcore_axis_name="s")
_SC_PARAMS = pltpu.CompilerParams(use_tc_tiling_on_sc=False,
                                  needs_layout_passes=False)


def _tile_rows(wid):
  rb = wid * 78 + jnp.maximum(wid - 28, 0)
  nblk = 78 + (wid >= 28).astype(jnp.int32)
  return rb, nblk


def _make_deg():
  mesh = plsc.VectorSubcoreMesh(**_MESH)

  @functools.partial(
      pl.kernel,
      out_type=jax.ShapeDtypeStruct((2, NDEG), jnp.float32),
      mesh=mesh,
      compiler_params=_SC_PARAMS,
      scratch_types=[
          pltpu.VMEM((RPT, 128), jnp.int32),
          pltpu.VMEM((128,), jnp.float32),
          pltpu.VMEM((NDEG // 16,), jnp.float32),
          pltpu.VMEM_SHARED((NDEG,), jnp.float32),
          pltpu.SemaphoreType.DMA,
      ],
  )
  def deg_kernel(dst_hbm, out_hbm, idx_v, ones_v, zero_v, acc, dsem):
    c = lax.axis_index("c")
    s = lax.axis_index("s")
    wid = c * 16 + s
    rb, nblk = _tile_rows(wid)
    npt = NDEG // 16
    one16 = jnp.full((16,), 1.0, jnp.float32)
    zero16 = jnp.zeros((16,), jnp.float32)
    for i in range(8):
      ones_v[pl.ds(i * 16, 16)] = one16

    def zbody(i, _):
      zero_v[pl.ds(i * 16, 16)] = zero16
      return 0

    lax.fori_loop(0, npt // 16, zbody, 0)
    pltpu.sync_copy(zero_v, acc.at[pl.ds(s * npt, npt)])
    pltpu.sync_copy(dst_hbm.at[pl.ds(rb, RPT)], idx_v)
    plsc.subcore_barrier()

    def body(j, _):
      pltpu.async_copy(ones_v, acc.at[idx_v.at[j]], dsem, add=True)
      return 0

    lax.fori_loop(0, nblk, body, 0)

    def drain(j, _):
      pltpu.make_async_copy(ones_v, acc.at[idx_v.at[j]], dsem).wait()
      return 0

    lax.fori_loop(0, nblk, drain, 0)
    plsc.subcore_barrier()
    pltpu.sync_copy(acc.at[pl.ds(s * npt, npt)],
                    out_hbm.at[c, pl.ds(s * npt, npt)])

  return deg_kernel


def _make_pass(d, scaled=False):
  mesh = plsc.VectorSubcoreMesh(**_MESH)
  scratch = [
      pltpu.VMEM((RPT, 128), jnp.int32),
      pltpu.VMEM((RPT, 128), jnp.int32),
      pltpu.VMEM((128, d), jnp.float32),
      pltpu.VMEM((128, d), jnp.float32),
      pltpu.VMEM_SHARED((N, d), jnp.float32),
      pltpu.VMEM_SHARED((N, d), jnp.float32),
      pltpu.SemaphoreType.DMA,
      pltpu.SemaphoreType.DMA,
  ]
  if scaled:
    scratch += [
        pltpu.VMEM((NPT, d), jnp.float32),
        pltpu.VMEM((NPT + 7, ), jnp.float32),
    ]

  def pass_body(hs_hbm, dinv_hbm, src_hbm, dst_hbm, zz_hbm, out_hbm,
                sidx, didx, rows0, rows1, table, acc, sem0, sem1,
                tmp=None, dvec=None):
    c = lax.axis_index("c")
    s = lax.axis_index("s")
    wid = c * 16 + s
    rb, nblk = _tile_rows(wid)
    if not scaled:
      pltpu.sync_copy(hs_hbm.at[pl.ds(s * NPT, NPT), pl.ds(0, d)],
                      table.at[pl.ds(s * NPT, NPT)])
    else:
      pltpu.sync_copy(hs_hbm.at[pl.ds(s * NPT, NPT), pl.ds(0, d)], tmp)
      b8 = s * NPT // 8 * 8
      off = s * NPT - b8
      pltpu.sync_copy(dinv_hbm.at[pl.ds(b8, NPT + 7)], dvec)

      def scale_row(i, _):
        dv = plsc.load_gather(
            dvec, [jnp.zeros((16,), jnp.int32) + (off + i)])
        tmp[i] = tmp[i] * dv
        return 0

      lax.fori_loop(0, NPT, scale_row, 0)
      pltpu.sync_copy(tmp, table.at[pl.ds(s * NPT, NPT)])
    pltpu.sync_copy(zz_hbm, acc.at[pl.ds(s * NPT, NPT)])
    pltpu.sync_copy(src_hbm.at[pl.ds(rb, RPT)], sidx)
    pltpu.sync_copy(dst_hbm.at[pl.ds(rb, RPT)], didx)
    plsc.subcore_barrier()

    pltpu.async_copy(table.at[sidx.at[0]], rows0, sem0)

    def body(i, _):
      b0 = 2 * i
      b1 = 2 * i + 1
      pltpu.async_copy(table.at[sidx.at[b1]], rows1, sem1)
      pltpu.make_async_copy(table.at[sidx.at[b0]], rows0, sem0).wait()
      pltpu.sync_copy(rows0, acc.at[didx.at[b0]], add=True)

      @pl.when(b0 + 2 < nblk)
      def _():
        pltpu.async_copy(table.at[sidx.at[b0 + 2]], rows0, sem0)

      pltpu.make_async_copy(table.at[sidx.at[b1]], rows1, sem1).wait()
      pltpu.sync_copy(rows1, acc.at[didx.at[b1]], add=True)
      return 0

    lax.fori_loop(0, 39, body, 0)

    @pl.when(nblk == RPT)
    def _():
      pltpu.make_async_copy(table.at[sidx.at[RPT - 1]], rows0, sem0).wait()
      pltpu.sync_copy(rows0, acc.at[didx.at[RPT - 1]], add=True)

    plsc.subcore_barrier()
    pltpu.sync_copy(acc.at[pl.ds(s * NPT, NPT)],
                    out_hbm.at[c, pl.ds(s * NPT, NPT), pl.ds(0, d)])

  kern = functools.partial(
      pl.kernel,
      out_type=jax.ShapeDtypeStruct((2, N, 128), jnp.float32),
      mesh=mesh,
      compiler_params=_SC_PARAMS,
      scratch_types=scratch,
  )
  if scaled:
    return kern(pass_body)

  def body_unscaled(hs_hbm, src_hbm, dst_hbm, zz_hbm, out_hbm, *rest):
    pass_body(hs_hbm, None, src_hbm, dst_hbm, zz_hbm, out_hbm, *rest)

  return kern(body_unscaled)


_deg_call = _make_deg()
_pass16 = _make_pass(D1, scaled=True)
_pass40 = _make_pass(D2)


def _stage_a_body(x_ref, w_ref, o_ref):
  o_ref[:, :D1] = jnp.dot(x_ref[:, :], w_ref[:, :],
                          preferred_element_type=jnp.float32)


def _stage_a(x, w1):
  return pl.pallas_call(
      _stage_a_body,
      out_shape=jax.ShapeDtypeStruct((N, 128), jnp.float32),
  )(x, w1)


def _stage_b_body(p_ref, h_ref, d_ref, b1_ref, w2_ref, o_ref):
  dcol = d_ref[:, :]
  t = ((p_ref[0, :, :D1] + p_ref[1, :, :D1] + h_ref[:, :D1] * dcol) * dcol
       + b1_ref[:, :])
  t = jnp.maximum(t, 0.0)
  o_ref[:, :D2] = jnp.dot(t, w2_ref[:, :],
                          preferred_element_type=jnp.float32) * dcol


def _stage_b(p1, hs1, dinv_col, b1r, w2):
  return pl.pallas_call(
      _stage_b_body,
      out_shape=jax.ShapeDtypeStruct((N, 128), jnp.float32),
  )(p1, hs1, dinv_col, b1r, w2)


def _stage_c_body(p_ref, h_ref, d_ref, b2_ref, o_ref):
  logits = ((p_ref[0, :, :D2] + p_ref[1, :, :D2] + h_ref[:, :D2])
            * d_ref[:, :] + b2_ref[:, :])
  m = jnp.max(logits, axis=1, keepdims=True)
  e = jnp.exp(logits - m)
  ssum = jnp.sum(e, axis=1, keepdims=True)
  o_ref[:, :] = logits - m - jnp.log(ssum)


def _stage_c(p2, hs2, dinv_col, b2r):
  return pl.pallas_call(
      _stage_c_body,
      out_shape=jax.ShapeDtypeStruct((N, D2), jnp.float32),
  )(p2, hs2, dinv_col, b2r)


def kernel(x, edge_index, W1, b1, W2, b2):
  ei = edge_index.astype(jnp.int32).reshape(2, EROWS, 128)
  src2d = ei[0]
  dst2d = ei[1]

  deg_p = _deg_call(dst2d)
  deg = deg_p[0] + deg_p[1] + 1.0
  dinv = lax.rsqrt(deg)
  dinv_col = dinv[:N].reshape(N, 1)
  zz16 = jnp.zeros((NPT, D1), jnp.float32)
  zz40 = jnp.zeros((NPT, D2), jnp.float32)

  h1 = _stage_a(x, W1)
  p1 = _pass16(h1, dinv, src2d, dst2d, zz16)

  b1r = b1.reshape(1, D1)
  b2r = b2.reshape(1, D2)

  hs2 = _stage_b(p1, h1, dinv_col, b1r, W2)
  p2 = _pass40(hs2, src2d, dst2d, zz40)
  return _stage_c(p2, hs2, dinv_col, b2r)


# speedup vs baseline: 1.3031x; 1.0865x over previous
import functools

import jax
import jax.numpy as jnp
from jax import lax
from jax.experimental import pallas as pl
from jax.experimental.pallas import tpu as pltpu
from jax.experimental.pallas import tpu_sc as plsc

N = 10000
NDEG = 10240
EROWS = 2500
RPT = 79
NPT = N // 16
D1 = 16
D2 = 40
RBLK = 2000

_MESH = dict(core_axis_name="c", subcore_axis_name="s")
_SC_PARAMS = pltpu.CompilerParams(use_tc_tiling_on_sc=False,
                                  needs_layout_passes=False)


def _tile_rows(wid):
  rb = wid * 78 + jnp.maximum(wid - 28, 0)
  nblk = 78 + (wid >= 28).astype(jnp.int32)
  return rb, nblk


def _make_deg():
  mesh = plsc.VectorSubcoreMesh(**_MESH)

  @functools.partial(
      pl.kernel,
      out_type=jax.ShapeDtypeStruct((2, NDEG), jnp.float32),
      mesh=mesh,
      compiler_params=_SC_PARAMS,
      scratch_types=[
          pltpu.VMEM((RPT, 128), jnp.int32),
          pltpu.VMEM((128,), jnp.float32),
          pltpu.VMEM((NDEG // 16,), jnp.float32),
          pltpu.VMEM_SHARED((NDEG,), jnp.float32),
          pltpu.SemaphoreType.DMA,
      ],
  )
  def deg_kernel(edges_hbm, out_hbm, idx_v, ones_v, zero_v, acc, dsem):
    c = lax.axis_index("c")
    s = lax.axis_index("s")
    wid = c * 16 + s
    rb, nblk = _tile_rows(wid)
    npt = NDEG // 16
    one16 = jnp.full((16,), 1.0, jnp.float32)
    zero16 = jnp.zeros((16,), jnp.float32)
    for i in range(8):
      ones_v[pl.ds(i * 16, 16)] = one16

    def zbody(i, _):
      zero_v[pl.ds(i * 16, 16)] = zero16
      return 0

    lax.fori_loop(0, npt // 16, zbody, 0)
    pltpu.sync_copy(zero_v, acc.at[pl.ds(s * npt, npt)])
    pltpu.sync_copy(edges_hbm.at[1, pl.ds(rb, RPT)], idx_v)
    plsc.subcore_barrier()

    def body(j, _):
      pltpu.async_copy(ones_v, acc.at[idx_v.at[j]], dsem, add=True)
      return 0

    lax.fori_loop(0, nblk, body, 0)

    def drain(j, _):
      pltpu.make_async_copy(ones_v, acc.at[idx_v.at[j]], dsem).wait()
      return 0

    lax.fori_loop(0, nblk, drain, 0)
    plsc.subcore_barrier()
    pltpu.sync_copy(acc.at[pl.ds(s * npt, npt)],
                    out_hbm.at[c, pl.ds(s * npt, npt)])

  return deg_kernel


def _make_pass(d, scaled=False):
  mesh = plsc.VectorSubcoreMesh(**_MESH)
  scratch = [
      pltpu.VMEM((RPT, 128), jnp.int32),
      pltpu.VMEM((RPT, 128), jnp.int32),
      pltpu.VMEM((128, d), jnp.float32),
      pltpu.VMEM((128, d), jnp.float32),
      pltpu.VMEM_SHARED((N, d), jnp.float32),
      pltpu.VMEM_SHARED((N, d), jnp.float32),
      pltpu.SemaphoreType.DMA,
      pltpu.SemaphoreType.DMA,
  ]
  if scaled:
    scratch += [
        pltpu.VMEM((NPT, d), jnp.float32),
        pltpu.VMEM((NPT + 7, ), jnp.float32),
    ]

  def pass_body(hs_hbm, dinv_hbm, edges_hbm, zz_hbm, out_hbm,
                sidx, didx, rows0, rows1, table, acc, sem0, sem1,
                tmp=None, dvec=None):
    c = lax.axis_index("c")
    s = lax.axis_index("s")
    wid = c * 16 + s
    rb, nblk = _tile_rows(wid)
    if not scaled:
      pltpu.sync_copy(hs_hbm.at[pl.ds(s * NPT, NPT), pl.ds(0, d)],
                      table.at[pl.ds(s * NPT, NPT)])
    else:
      pltpu.sync_copy(hs_hbm.at[pl.ds(s * NPT, NPT), pl.ds(0, d)], tmp)
      b8 = s * NPT // 8 * 8
      off = s * NPT - b8
      pltpu.sync_copy(dinv_hbm.at[pl.ds(b8, NPT + 7)], dvec)

      def scale_row(i, _):
        dv = plsc.load_gather(
            dvec, [jnp.zeros((16,), jnp.int32) + (off + i)])
        tmp[i] = tmp[i] * dv
        return 0

      lax.fori_loop(0, NPT, scale_row, 0)
      pltpu.sync_copy(tmp, table.at[pl.ds(s * NPT, NPT)])
    pltpu.sync_copy(zz_hbm, acc.at[pl.ds(s * NPT, NPT)])
    pltpu.sync_copy(edges_hbm.at[0, pl.ds(rb, RPT)], sidx)
    pltpu.sync_copy(edges_hbm.at[1, pl.ds(rb, RPT)], didx)
    plsc.subcore_barrier()

    pltpu.async_copy(table.at[sidx.at[0]], rows0, sem0)

    def body(i, _):
      b0 = 2 * i
      b1 = 2 * i + 1
      pltpu.async_copy(table.at[sidx.at[b1]], rows1, sem1)
      pltpu.make_async_copy(table.at[sidx.at[b0]], rows0, sem0).wait()
      pltpu.sync_copy(rows0, acc.at[didx.at[b0]], add=True)

      @pl.when(b0 + 2 < nblk)
      def _():
        pltpu.async_copy(table.at[sidx.at[b0 + 2]], rows0, sem0)

      pltpu.make_async_copy(table.at[sidx.at[b1]], rows1, sem1).wait()
      pltpu.sync_copy(rows1, acc.at[didx.at[b1]], add=True)
      return 0

    lax.fori_loop(0, 39, body, 0)

    @pl.when(nblk == RPT)
    def _():
      pltpu.make_async_copy(table.at[sidx.at[RPT - 1]], rows0, sem0).wait()
      pltpu.sync_copy(rows0, acc.at[didx.at[RPT - 1]], add=True)

    plsc.subcore_barrier()
    pltpu.sync_copy(acc.at[pl.ds(s * NPT, NPT)],
                    out_hbm.at[c, pl.ds(s * NPT, NPT), pl.ds(0, d)])

  kern = functools.partial(
      pl.kernel,
      out_type=jax.ShapeDtypeStruct((2, N, 128), jnp.float32),
      mesh=mesh,
      compiler_params=_SC_PARAMS,
      scratch_types=scratch,
  )
  if scaled:
    return kern(pass_body)

  def body_unscaled(hs_hbm, edges_hbm, zz_hbm, out_hbm, *rest):
    pass_body(hs_hbm, None, edges_hbm, zz_hbm, out_hbm, *rest)

  return kern(body_unscaled)


_deg_call = _make_deg()
_pass16 = _make_pass(D1, scaled=True)
_pass40 = _make_pass(D2)


def _stage_a_body(x_ref, w_ref, o_ref):
  o_ref[:, :D1] = jnp.dot(x_ref[:, :], w_ref[:, :],
                          preferred_element_type=jnp.float32)


def _stage_a(x, w1):
  return pl.pallas_call(
      _stage_a_body,
      out_shape=jax.ShapeDtypeStruct((N, 128), jnp.float32),
  )(x, w1)


def _stage_b_body(p_ref, h_ref, d_ref, b1_ref, w2_ref, o_ref):
  dcol = d_ref[:, :]
  t = ((p_ref[0, :, :D1] + p_ref[1, :, :D1] + h_ref[:, :D1] * dcol) * dcol
       + b1_ref[:, :])
  t = jnp.maximum(t, 0.0)
  o_ref[:, :D2] = jnp.dot(t, w2_ref[:, :],
                          preferred_element_type=jnp.float32) * dcol


def _stage_b(p1, hs1, dinv_col, b1r, w2):
  return pl.pallas_call(
      _stage_b_body,
      out_shape=jax.ShapeDtypeStruct((N, 128), jnp.float32),
  )(p1, hs1, dinv_col, b1r, w2)


def _stage_c_body(p_ref, h_ref, d_ref, b2_ref, o_ref):
  logits = ((p_ref[0, :, :D2] + p_ref[1, :, :D2] + h_ref[:, :D2])
            * d_ref[:, :] + b2_ref[:, :])
  m = jnp.max(logits, axis=1, keepdims=True)
  e = jnp.exp(logits - m)
  ssum = jnp.sum(e, axis=1, keepdims=True)
  o_ref[:, :] = logits - m - jnp.log(ssum)


def _stage_c(p2, hs2, dinv_col, b2r):
  return pl.pallas_call(
      _stage_c_body,
      out_shape=jax.ShapeDtypeStruct((N, D2), jnp.float32),
  )(p2, hs2, dinv_col, b2r)


def kernel(x, edge_index, W1, b1, W2, b2):
  edges3 = edge_index.astype(jnp.int32).reshape(2, EROWS, 128)

  deg_p = _deg_call(edges3)
  deg = deg_p[0] + deg_p[1] + 1.0
  dinv = lax.rsqrt(deg)
  dinv_col = dinv[:N].reshape(N, 1)
  zz16 = jnp.zeros((NPT, D1), jnp.float32)
  zz40 = jnp.zeros((NPT, D2), jnp.float32)

  h1 = _stage_a(x, W1)
  p1 = _pass16(h1, dinv, edges3, zz16)

  b1r = b1.reshape(1, D1)
  b2r = b2.reshape(1, D2)

  hs2 = _stage_b(p1, h1, dinv_col, b1r, W2)
  p2 = _pass40(hs2, edges3, zz40)
  return _stage_c(p2, hs2, dinv_col, b2r)


# speedup vs baseline: 1.3063x; 1.0025x over previous
import functools

import jax
import jax.numpy as jnp
from jax import lax
from jax.experimental import pallas as pl
from jax.experimental.pallas import tpu as pltpu
from jax.experimental.pallas import tpu_sc as plsc

N = 10000
NDEG = 10240
EROWS = 2500
RPT = 79
NPT = N // 16
D1 = 16
D2 = 40
RBLK = 2000

_MESH = dict(core_axis_name="c", subcore_axis_name="s")
_SC_PARAMS = pltpu.CompilerParams(use_tc_tiling_on_sc=False,
                                  needs_layout_passes=False)


def _tile_rows(wid):
  rb = wid * 78 + jnp.maximum(wid - 28, 0)
  nblk = 78 + (wid >= 28).astype(jnp.int32)
  return rb, nblk


def _make_deg():
  mesh = plsc.VectorSubcoreMesh(**_MESH)

  @functools.partial(
      pl.kernel,
      out_type=jax.ShapeDtypeStruct((2, NDEG), jnp.float32),
      mesh=mesh,
      compiler_params=_SC_PARAMS,
      scratch_types=[
          pltpu.VMEM((RPT, 128), jnp.int32),
          pltpu.VMEM((128,), jnp.float32),
          pltpu.VMEM((NDEG // 16,), jnp.float32),
          pltpu.VMEM_SHARED((NDEG,), jnp.float32),
          pltpu.SemaphoreType.DMA,
      ],
  )
  def deg_kernel(edges_hbm, out_hbm, idx_v, ones_v, zero_v, acc, dsem):
    c = lax.axis_index("c")
    s = lax.axis_index("s")
    wid = c * 16 + s
    rb, nblk = _tile_rows(wid)
    npt = NDEG // 16
    one16 = jnp.full((16,), 1.0, jnp.float32)
    zero16 = jnp.zeros((16,), jnp.float32)
    for i in range(8):
      ones_v[pl.ds(i * 16, 16)] = one16

    def zbody(i, _):
      zero_v[pl.ds(i * 16, 16)] = zero16
      return 0

    lax.fori_loop(0, npt // 16, zbody, 0)
    pltpu.sync_copy(zero_v, acc.at[pl.ds(s * npt, npt)])
    pltpu.sync_copy(edges_hbm.at[1, pl.ds(rb, RPT)], idx_v)
    plsc.subcore_barrier()

    def body(j, _):
      pltpu.async_copy(ones_v, acc.at[idx_v.at[j]], dsem, add=True)
      return 0

    lax.fori_loop(0, nblk, body, 0)

    def drain(j, _):
      pltpu.make_async_copy(ones_v, acc.at[idx_v.at[j]], dsem).wait()
      return 0

    lax.fori_loop(0, nblk, drain, 0)
    plsc.subcore_barrier()
    pltpu.sync_copy(acc.at[pl.ds(s * npt, npt)],
                    out_hbm.at[c, pl.ds(s * npt, npt)])

  return deg_kernel


def _make_pass(d, scaled=False):
  mesh = plsc.VectorSubcoreMesh(**_MESH)
  scratch = [
      pltpu.VMEM((RPT, 128), jnp.int32),
      pltpu.VMEM((RPT, 128), jnp.int32),
      pltpu.VMEM((128, d), jnp.float32),
      pltpu.VMEM((128, d), jnp.float32),
      pltpu.VMEM_SHARED((N, d), jnp.float32),
      pltpu.VMEM_SHARED((N, d), jnp.float32),
      pltpu.SemaphoreType.DMA,
      pltpu.SemaphoreType.DMA,
  ]
  if scaled:
    scratch += [
        pltpu.VMEM((NPT, d), jnp.float32),
        pltpu.VMEM((NPT + 7, ), jnp.float32),
    ]

  def pass_body(hs_hbm, dinv_hbm, edges_hbm, zz_hbm, out_hbm,
                sidx, didx, rows0, rows1, table, acc, sem0, sem1,
                tmp=None, dvec=None):
    c = lax.axis_index("c")
    s = lax.axis_index("s")
    wid = c * 16 + s
    rb, nblk = _tile_rows(wid)
    if not scaled:
      pltpu.sync_copy(hs_hbm.at[pl.ds(s * NPT, NPT), pl.ds(0, d)],
                      table.at[pl.ds(s * NPT, NPT)])
    else:
      pltpu.sync_copy(hs_hbm.at[pl.ds(s * NPT, NPT), pl.ds(0, d)], tmp)
      b8 = s * NPT // 8 * 8
      off = s * NPT - b8
      pltpu.sync_copy(dinv_hbm.at[pl.ds(b8, NPT + 7)], dvec)

      def scale_row(i, _):
        dv = plsc.load_gather(
            dvec, [jnp.zeros((16,), jnp.int32) + (off + i)])
        tmp[i] = tmp[i] * dv
        return 0

      lax.fori_loop(0, NPT, scale_row, 0)
      pltpu.sync_copy(tmp, table.at[pl.ds(s * NPT, NPT)])
    pltpu.sync_copy(zz_hbm, acc.at[pl.ds(s * NPT, NPT)])
    pltpu.sync_copy(edges_hbm.at[0, pl.ds(rb, RPT)], sidx)
    pltpu.sync_copy(edges_hbm.at[1, pl.ds(rb, RPT)], didx)
    plsc.subcore_barrier()

    pltpu.async_copy(table.at[sidx.at[0]], rows0, sem0)

    def body(i, _):
      b0 = 2 * i
      b1 = 2 * i + 1
      pltpu.async_copy(table.at[sidx.at[b1]], rows1, sem1)
      pltpu.make_async_copy(table.at[sidx.at[b0]], rows0, sem0).wait()
      pltpu.sync_copy(rows0, acc.at[didx.at[b0]], add=True)

      @pl.when(b0 + 2 < nblk)
      def _():
        pltpu.async_copy(table.at[sidx.at[b0 + 2]], rows0, sem0)

      pltpu.make_async_copy(table.at[sidx.at[b1]], rows1, sem1).wait()
      pltpu.sync_copy(rows1, acc.at[didx.at[b1]], add=True)
      return 0

    lax.fori_loop(0, 39, body, 0)

    @pl.when(nblk == RPT)
    def _():
      pltpu.make_async_copy(table.at[sidx.at[RPT - 1]], rows0, sem0).wait()
      pltpu.sync_copy(rows0, acc.at[didx.at[RPT - 1]], add=True)

    plsc.subcore_barrier()
    pltpu.sync_copy(acc.at[pl.ds(s * NPT, NPT)],
                    out_hbm.at[pl.ds(s * NPT, NPT), pl.ds(c * d, d)])

  kern = functools.partial(
      pl.kernel,
      out_type=jax.ShapeDtypeStruct((N, 128), jnp.float32),
      mesh=mesh,
      compiler_params=_SC_PARAMS,
      scratch_types=scratch,
  )
  if scaled:
    return kern(pass_body)

  def body_unscaled(hs_hbm, edges_hbm, zz_hbm, out_hbm, *rest):
    pass_body(hs_hbm, None, edges_hbm, zz_hbm, out_hbm, *rest)

  return kern(body_unscaled)


_deg_call = _make_deg()
_pass16 = _make_pass(D1, scaled=True)
_pass40 = _make_pass(D2)


def _stage_a_body(x_ref, w_ref, o_ref):
  o_ref[:, :D1] = jnp.dot(x_ref[:, :], w_ref[:, :],
                          preferred_element_type=jnp.float32)


def _stage_a(x, w1):
  return pl.pallas_call(
      _stage_a_body,
      out_shape=jax.ShapeDtypeStruct((N, 128), jnp.float32),
  )(x, w1)


def _stage_b_body(p_ref, h_ref, d_ref, b1_ref, w2_ref, o_ref):
  dcol = d_ref[:, :]
  t = ((p_ref[:, :D1] + p_ref[:, D1:2 * D1] + h_ref[:, :D1] * dcol) * dcol
       + b1_ref[:, :])
  t = jnp.maximum(t, 0.0)
  o_ref[:, :D2] = jnp.dot(t, w2_ref[:, :],
                          preferred_element_type=jnp.float32) * dcol


def _stage_b(p1, hs1, dinv_col, b1r, w2):
  return pl.pallas_call(
      _stage_b_body,
      out_shape=jax.ShapeDtypeStruct((N, 128), jnp.float32),
  )(p1, hs1, dinv_col, b1r, w2)


def _stage_c_body(p_ref, h_ref, d_ref, b2_ref, o_ref):
  logits = ((p_ref[:, :D2] + p_ref[:, D2:2 * D2] + h_ref[:, :D2])
            * d_ref[:, :] + b2_ref[:, :])
  m = jnp.max(logits, axis=1, keepdims=True)
  e = jnp.exp(logits - m)
  ssum = jnp.sum(e, axis=1, keepdims=True)
  o_ref[:, :] = logits - m - jnp.log(ssum)


def _stage_c(p2, hs2, dinv_col, b2r):
  return pl.pallas_call(
      _stage_c_body,
      out_shape=jax.ShapeDtypeStruct((N, D2), jnp.float32),
  )(p2, hs2, dinv_col, b2r)


def kernel(x, edge_index, W1, b1, W2, b2):
  edges3 = edge_index.astype(jnp.int32).reshape(2, EROWS, 128)

  deg_p = _deg_call(edges3)
  deg = deg_p[0] + deg_p[1] + 1.0
  dinv = lax.rsqrt(deg)
  dinv_col = dinv[:N].reshape(N, 1)
  zz16 = jnp.zeros((NPT, D1), jnp.float32)
  zz40 = jnp.zeros((NPT, D2), jnp.float32)

  h1 = _stage_a(x, W1)
  p1 = _pass16(h1, dinv, edges3, zz16)

  b1r = b1.reshape(1, D1)
  b2r = b2.reshape(1, D2)

  hs2 = _stage_b(p1, h1, dinv_col, b1r, W2)
  p2 = _pass40(hs2, edges3, zz40)
  return _stage_c(p2, hs2, dinv_col, b2r)


# speedup vs baseline: 1.3533x; 1.0360x over previous
import functools

import jax
import jax.numpy as jnp
from jax import lax
from jax.experimental import pallas as pl
from jax.experimental.pallas import tpu as pltpu
from jax.experimental.pallas import tpu_sc as plsc

N = 10000
NDEG = 10240
EROWS = 2500
RPT = 79
NPT = N // 16
D1 = 16
D2 = 40
RBLK = 2000

_MESH = dict(core_axis_name="c", subcore_axis_name="s")
_SC_PARAMS = pltpu.CompilerParams(use_tc_tiling_on_sc=False,
                                  needs_layout_passes=False)


def _tile_rows(wid):
  rb = wid * 78 + jnp.maximum(wid - 28, 0)
  nblk = 78 + (wid >= 28).astype(jnp.int32)
  return rb, nblk


def _make_deg():
  mesh = plsc.VectorSubcoreMesh(**_MESH)

  @functools.partial(
      pl.kernel,
      out_type=jax.ShapeDtypeStruct((2, NDEG), jnp.float32),
      mesh=mesh,
      compiler_params=_SC_PARAMS,
      scratch_types=[
          pltpu.VMEM((RPT, 128), jnp.int32),
          pltpu.VMEM((128,), jnp.float32),
          pltpu.VMEM((NDEG // 16,), jnp.float32),
          pltpu.VMEM_SHARED((NDEG,), jnp.float32),
          pltpu.SemaphoreType.DMA,
      ],
  )
  def deg_kernel(edges_hbm, out_hbm, idx_v, ones_v, zero_v, acc, dsem):
    c = lax.axis_index("c")
    s = lax.axis_index("s")
    wid = c * 16 + s
    rb, nblk = _tile_rows(wid)
    npt = NDEG // 16
    one16 = jnp.full((16,), 1.0, jnp.float32)
    zero16 = jnp.zeros((16,), jnp.float32)
    for i in range(8):
      ones_v[pl.ds(i * 16, 16)] = one16

    def zbody(i, _):
      zero_v[pl.ds(i * 16, 16)] = zero16
      return 0

    lax.fori_loop(0, npt // 16, zbody, 0)
    pltpu.sync_copy(zero_v, acc.at[pl.ds(s * npt, npt)])
    pltpu.sync_copy(edges_hbm.at[1, pl.ds(rb, RPT)], idx_v)
    plsc.subcore_barrier()

    def body(j, _):
      pltpu.async_copy(ones_v, acc.at[idx_v.at[j]], dsem, add=True)
      return 0

    lax.fori_loop(0, nblk, body, 0)

    def drain(j, _):
      pltpu.make_async_copy(ones_v, acc.at[idx_v.at[j]], dsem).wait()
      return 0

    lax.fori_loop(0, nblk, drain, 0)
    plsc.subcore_barrier()
    pltpu.sync_copy(acc.at[pl.ds(s * npt, npt)],
                    out_hbm.at[c, pl.ds(s * npt, npt)])

  return deg_kernel


def _make_pass(d, scaled=False):
  mesh = plsc.VectorSubcoreMesh(**_MESH)
  scratch = [
      pltpu.VMEM((RPT, 128), jnp.int32),
      pltpu.VMEM((RPT, 128), jnp.int32),
      pltpu.VMEM((128, d), jnp.float32),
      pltpu.VMEM((128, d), jnp.float32),
      pltpu.VMEM_SHARED((N, d), jnp.float32),
      pltpu.VMEM_SHARED((N, d), jnp.float32),
      pltpu.SemaphoreType.DMA,
      pltpu.SemaphoreType.DMA,
  ]
  if scaled:
    scratch += [
        pltpu.VMEM((NPT, d), jnp.float32),
        pltpu.VMEM((NPT + 7, ), jnp.float32),
    ]

  def pass_body(hs_hbm, dinv_hbm, edges_hbm, zz_hbm, out_hbm,
                sidx, didx, rows0, rows1, table, acc, sem0, sem1,
                tmp=None, dvec=None):
    c = lax.axis_index("c")
    s = lax.axis_index("s")
    wid = c * 16 + s
    rb, nblk = _tile_rows(wid)
    pltpu.async_copy(zz_hbm, acc.at[pl.ds(s * NPT, NPT)], sem0)
    pltpu.async_copy(edges_hbm.at[0, pl.ds(rb, RPT)], sidx, sem1)
    pltpu.async_copy(edges_hbm.at[1, pl.ds(rb, RPT)], didx, sem1)
    if not scaled:
      pltpu.async_copy(hs_hbm.at[pl.ds(s * NPT, NPT), pl.ds(0, d)],
                       table.at[pl.ds(s * NPT, NPT)], sem0)
      pltpu.make_async_copy(
          hs_hbm.at[pl.ds(s * NPT, NPT), pl.ds(0, d)],
          table.at[pl.ds(s * NPT, NPT)], sem0).wait()
    else:
      pltpu.async_copy(hs_hbm.at[pl.ds(s * NPT, NPT), pl.ds(0, d)], tmp,
                       sem0)
      b8 = s * NPT // 8 * 8
      off = s * NPT - b8
      pltpu.sync_copy(dinv_hbm.at[pl.ds(b8, NPT + 7)], dvec)
      pltpu.make_async_copy(hs_hbm.at[pl.ds(s * NPT, NPT), pl.ds(0, d)],
                            tmp, sem0).wait()

      def scale_row(i, _):
        dv = plsc.load_gather(
            dvec, [jnp.zeros((16,), jnp.int32) + (off + i)])
        tmp[i] = tmp[i] * dv
        return 0

      lax.fori_loop(0, NPT, scale_row, 0)
      pltpu.sync_copy(tmp, table.at[pl.ds(s * NPT, NPT)])
    pltpu.make_async_copy(zz_hbm, acc.at[pl.ds(s * NPT, NPT)], sem0).wait()
    pltpu.make_async_copy(edges_hbm.at[0, pl.ds(rb, RPT)], sidx,
                          sem1).wait()
    pltpu.make_async_copy(edges_hbm.at[1, pl.ds(rb, RPT)], didx,
                          sem1).wait()
    plsc.subcore_barrier()

    pltpu.async_copy(table.at[sidx.at[0]], rows0, sem0)

    def body(i, _):
      b0 = 2 * i
      b1 = 2 * i + 1
      pltpu.async_copy(table.at[sidx.at[b1]], rows1, sem1)
      pltpu.make_async_copy(table.at[sidx.at[b0]], rows0, sem0).wait()
      pltpu.sync_copy(rows0, acc.at[didx.at[b0]], add=True)

      @pl.when(b0 + 2 < nblk)
      def _():
        pltpu.async_copy(table.at[sidx.at[b0 + 2]], rows0, sem0)

      pltpu.make_async_copy(table.at[sidx.at[b1]], rows1, sem1).wait()
      pltpu.sync_copy(rows1, acc.at[didx.at[b1]], add=True)
      return 0

    lax.fori_loop(0, 39, body, 0)

    @pl.when(nblk == RPT)
    def _():
      pltpu.make_async_copy(table.at[sidx.at[RPT - 1]], rows0, sem0).wait()
      pltpu.sync_copy(rows0, acc.at[didx.at[RPT - 1]], add=True)

    plsc.subcore_barrier()
    pltpu.sync_copy(acc.at[pl.ds(s * NPT, NPT)],
                    out_hbm.at[pl.ds(s * NPT, NPT), pl.ds(c * d, d)])

  kern = functools.partial(
      pl.kernel,
      out_type=jax.ShapeDtypeStruct((N, 128), jnp.float32),
      mesh=mesh,
      compiler_params=_SC_PARAMS,
      scratch_types=scratch,
  )
  if scaled:
    return kern(pass_body)

  def body_unscaled(hs_hbm, edges_hbm, zz_hbm, out_hbm, *rest):
    pass_body(hs_hbm, None, edges_hbm, zz_hbm, out_hbm, *rest)

  return kern(body_unscaled)


_deg_call = _make_deg()
_pass16 = _make_pass(D1, scaled=True)
_pass40 = _make_pass(D2)


def _stage_a_body(x_ref, w_ref, o_ref):
  o_ref[:, :D1] = jnp.dot(x_ref[:, :], w_ref[:, :],
                          preferred_element_type=jnp.float32)


def _stage_a(x, w1):
  return pl.pallas_call(
      _stage_a_body,
      out_shape=jax.ShapeDtypeStruct((N, 128), jnp.float32),
  )(x, w1)


def _stage_b_body(p_ref, h_ref, d_ref, b1_ref, w2_ref, o_ref):
  dcol = d_ref[:, :]
  t = ((p_ref[:, :D1] + p_ref[:, D1:2 * D1] + h_ref[:, :D1] * dcol) * dcol
       + b1_ref[:, :])
  t = jnp.maximum(t, 0.0)
  o_ref[:, :D2] = jnp.dot(t, w2_ref[:, :],
                          preferred_element_type=jnp.float32) * dcol


def _stage_b(p1, hs1, dinv_col, b1r, w2):
  return pl.pallas_call(
      _stage_b_body,
      out_shape=jax.ShapeDtypeStruct((N, 128), jnp.float32),
  )(p1, hs1, dinv_col, b1r, w2)


def _stage_c_body(p_ref, h_ref, d_ref, b2_ref, o_ref):
  logits = ((p_ref[:, :D2] + p_ref[:, D2:2 * D2] + h_ref[:, :D2])
            * d_ref[:, :] + b2_ref[:, :])
  m = jnp.max(logits, axis=1, keepdims=True)
  e = jnp.exp(logits - m)
  ssum = jnp.sum(e, axis=1, keepdims=True)
  o_ref[:, :] = logits - m - jnp.log(ssum)


def _stage_c(p2, hs2, dinv_col, b2r):
  return pl.pallas_call(
      _stage_c_body,
      out_shape=jax.ShapeDtypeStruct((N, D2), jnp.float32),
  )(p2, hs2, dinv_col, b2r)


def kernel(x, edge_index, W1, b1, W2, b2):
  edges3 = edge_index.astype(jnp.int32).reshape(2, EROWS, 128)

  deg_p = _deg_call(edges3)
  deg = deg_p[0] + deg_p[1] + 1.0
  dinv = lax.rsqrt(deg)
  dinv_col = dinv[:N].reshape(N, 1)
  zz16 = jnp.zeros((NPT, D1), jnp.float32)
  zz40 = jnp.zeros((NPT, D2), jnp.float32)

  h1 = _stage_a(x, W1)
  p1 = _pass16(h1, dinv, edges3, zz16)

  b1r = b1.reshape(1, D1)
  b2r = b2.reshape(1, D2)

  hs2 = _stage_b(p1, h1, dinv_col, b1r, W2)
  p2 = _pass40(hs2, edges3, zz40)
  return _stage_c(p2, hs2, dinv_col, b2r)


# speedup vs baseline: 1.3991x; 1.0338x over previous
import functools

import jax
import jax.numpy as jnp
from jax import lax
from jax.experimental import pallas as pl
from jax.experimental.pallas import tpu as pltpu
from jax.experimental.pallas import tpu_sc as plsc

N = 10000
NDEG = 10240
EROWS = 2500
RPT = 79
NPT = N // 16
D1 = 16
D2 = 40
RBLK = 2000

_MESH = dict(core_axis_name="c", subcore_axis_name="s")
_SC_PARAMS = pltpu.CompilerParams(use_tc_tiling_on_sc=False,
                                  needs_layout_passes=False)


def _tile_rows(wid):
  rb = wid * 78 + jnp.maximum(wid - 28, 0)
  nblk = 78 + (wid >= 28).astype(jnp.int32)
  return rb, nblk


def _make_deg():
  mesh = plsc.VectorSubcoreMesh(**_MESH)

  @functools.partial(
      pl.kernel,
      out_type=jax.ShapeDtypeStruct((2, NDEG), jnp.float32),
      mesh=mesh,
      compiler_params=_SC_PARAMS,
      scratch_types=[
          pltpu.VMEM((RPT, 128), jnp.int32),
          pltpu.VMEM((128,), jnp.float32),
          pltpu.VMEM((NDEG // 16,), jnp.float32),
          pltpu.VMEM_SHARED((NDEG,), jnp.float32),
          pltpu.SemaphoreType.DMA,
      ],
  )
  def deg_kernel(edges_hbm, out_hbm, idx_v, ones_v, zero_v, acc, dsem):
    c = lax.axis_index("c")
    s = lax.axis_index("s")
    wid = c * 16 + s
    rb, nblk = _tile_rows(wid)
    npt = NDEG // 16
    one16 = jnp.full((16,), 1.0, jnp.float32)
    zero16 = jnp.zeros((16,), jnp.float32)
    for i in range(8):
      ones_v[pl.ds(i * 16, 16)] = one16

    def zbody(i, _):
      zero_v[pl.ds(i * 16, 16)] = zero16
      return 0

    lax.fori_loop(0, npt // 16, zbody, 0)
    pltpu.sync_copy(zero_v, acc.at[pl.ds(s * npt, npt)])
    pltpu.sync_copy(edges_hbm.at[1, pl.ds(rb, RPT)], idx_v)
    plsc.subcore_barrier()

    def body(j, _):
      pltpu.async_copy(ones_v, acc.at[idx_v.at[j]], dsem, add=True)
      return 0

    lax.fori_loop(0, nblk, body, 0)

    def drain(j, _):
      pltpu.make_async_copy(ones_v, acc.at[idx_v.at[j]], dsem).wait()
      return 0

    lax.fori_loop(0, nblk, drain, 0)
    plsc.subcore_barrier()
    pltpu.sync_copy(acc.at[pl.ds(s * npt, npt)],
                    out_hbm.at[c, pl.ds(s * npt, npt)])

  return deg_kernel


def _make_pass(d, scaled=False):
  mesh = plsc.VectorSubcoreMesh(**_MESH)
  scratch = [
      pltpu.VMEM((RPT, 128), jnp.int32),
      pltpu.VMEM((RPT, 128), jnp.int32),
      pltpu.VMEM((128, d), jnp.float32),
      pltpu.VMEM((128, d), jnp.float32),
      pltpu.VMEM_SHARED((N, d), jnp.float32),
      pltpu.VMEM_SHARED((N, d), jnp.float32),
      pltpu.SemaphoreType.DMA,
      pltpu.SemaphoreType.DMA,
  ]
  if scaled:
    scratch += [
        pltpu.VMEM((NPT, d), jnp.float32),
        pltpu.VMEM((NPT + 7, ), jnp.float32),
    ]

  def pass_body(hs_hbm, dinv_hbm, edges_hbm, zz_hbm, out_hbm,
                sidx, didx, rows0, rows1, table, acc, sem0, sem1,
                tmp=None, dvec=None):
    c = lax.axis_index("c")
    s = lax.axis_index("s")
    wid = c * 16 + s
    rb, nblk = _tile_rows(wid)
    pltpu.async_copy(zz_hbm, acc.at[pl.ds(s * NPT, NPT)], sem0)
    pltpu.async_copy(edges_hbm.at[0, pl.ds(rb, RPT)], sidx, sem1)
    pltpu.async_copy(edges_hbm.at[1, pl.ds(rb, RPT)], didx, sem1)
    if not scaled:
      pltpu.async_copy(hs_hbm.at[pl.ds(s * NPT, NPT), pl.ds(0, d)],
                       table.at[pl.ds(s * NPT, NPT)], sem0)
      pltpu.make_async_copy(
          hs_hbm.at[pl.ds(s * NPT, NPT), pl.ds(0, d)],
          table.at[pl.ds(s * NPT, NPT)], sem0).wait()
    else:
      pltpu.async_copy(hs_hbm.at[pl.ds(s * NPT, NPT), pl.ds(0, d)], tmp,
                       sem0)
      b8 = s * NPT // 8 * 8
      off = s * NPT - b8
      pltpu.sync_copy(dinv_hbm.at[pl.ds(b8, NPT + 7)], dvec)
      pltpu.make_async_copy(hs_hbm.at[pl.ds(s * NPT, NPT), pl.ds(0, d)],
                            tmp, sem0).wait()

      def scale_row(i, _):
        dv = plsc.load_gather(
            dvec, [jnp.zeros((16,), jnp.int32) + (off + i)])
        tmp[i] = tmp[i] * dv
        return 0

      lax.fori_loop(0, NPT, scale_row, 0)
      pltpu.sync_copy(tmp, table.at[pl.ds(s * NPT, NPT)])
    pltpu.make_async_copy(zz_hbm, acc.at[pl.ds(s * NPT, NPT)], sem0).wait()
    pltpu.make_async_copy(edges_hbm.at[0, pl.ds(rb, RPT)], sidx,
                          sem1).wait()
    pltpu.make_async_copy(edges_hbm.at[1, pl.ds(rb, RPT)], didx,
                          sem1).wait()
    plsc.subcore_barrier()

    pltpu.async_copy(table.at[sidx.at[0]], rows0, sem0)

    def body(i, _):
      b0 = 2 * i
      b1 = 2 * i + 1
      pltpu.async_copy(table.at[sidx.at[b1]], rows1, sem1)
      pltpu.make_async_copy(table.at[sidx.at[b0]], rows0, sem0).wait()
      pltpu.sync_copy(rows0, acc.at[didx.at[b0]], add=True)

      @pl.when(b0 + 2 < nblk)
      def _():
        pltpu.async_copy(table.at[sidx.at[b0 + 2]], rows0, sem0)

      pltpu.make_async_copy(table.at[sidx.at[b1]], rows1, sem1).wait()
      pltpu.sync_copy(rows1, acc.at[didx.at[b1]], add=True)
      return 0

    lax.fori_loop(0, 39, body, 0)

    @pl.when(nblk == RPT)
    def _():
      pltpu.make_async_copy(table.at[sidx.at[RPT - 1]], rows0, sem0).wait()
      pltpu.sync_copy(rows0, acc.at[didx.at[RPT - 1]], add=True)

    plsc.subcore_barrier()
    pltpu.sync_copy(acc.at[pl.ds(s * NPT, NPT)],
                    out_hbm.at[pl.ds(s * NPT, NPT), pl.ds(c * d, d)])

  kern = functools.partial(
      pl.kernel,
      out_type=jax.ShapeDtypeStruct((N, 128), jnp.float32),
      mesh=mesh,
      compiler_params=_SC_PARAMS,
      scratch_types=scratch,
  )
  if scaled:
    return kern(pass_body)

  def body_unscaled(hs_hbm, edges_hbm, zz_hbm, out_hbm, *rest):
    pass_body(hs_hbm, None, edges_hbm, zz_hbm, out_hbm, *rest)

  return kern(body_unscaled)


_deg_call = _make_deg()
_pass16 = _make_pass(D1, scaled=True)
_pass40 = _make_pass(D2)


def _stage_a_body(x_ref, w_ref, o_ref):
  o_ref[:, :D1] = jnp.dot(x_ref[:, :], w_ref[:, :],
                          preferred_element_type=jnp.float32)


def _stage_a(x, w1):
  return pl.pallas_call(
      _stage_a_body,
      out_shape=jax.ShapeDtypeStruct((N, 128), jnp.float32),
  )(x, w1)


def _stage_b_body(p_ref, h_ref, d_ref, b1_ref, w2_ref, o_ref):
  dcol = d_ref[:, :]
  t = ((p_ref[:, :D1] + p_ref[:, D1:2 * D1] + h_ref[:, :D1] * dcol) * dcol
       + b1_ref[:, :])
  t = jnp.maximum(t, 0.0)
  o_ref[:, :D2] = jnp.dot(t, w2_ref[:, :],
                          preferred_element_type=jnp.float32) * dcol


def _stage_b(p1, hs1, dinv_col, b1r, w2):
  return pl.pallas_call(
      _stage_b_body,
      out_shape=jax.ShapeDtypeStruct((N, 128), jnp.float32),
  )(p1, hs1, dinv_col, b1r, w2)


def _stage_c_body(p_ref, h_ref, d_ref, b2_ref, o_ref):
  logits = ((p_ref[:, :D2] + p_ref[:, D2:2 * D2] + h_ref[:, :D2])
            * d_ref[:, :] + b2_ref[:, :])
  m = jnp.max(logits, axis=1, keepdims=True)
  e = jnp.exp(logits - m)
  ssum = jnp.sum(e, axis=1, keepdims=True)
  o_ref[:, :] = (logits - m - jnp.log(ssum)).T


def _stage_c(p2, hs2, dinv_col, b2r):
  return pl.pallas_call(
      _stage_c_body,
      out_shape=jax.ShapeDtypeStruct((D2, N), jnp.float32),
  )(p2, hs2, dinv_col, b2r)


def kernel(x, edge_index, W1, b1, W2, b2):
  edges3 = edge_index.astype(jnp.int32).reshape(2, EROWS, 128)

  deg_p = _deg_call(edges3)
  deg = deg_p[0] + deg_p[1] + 1.0
  dinv = lax.rsqrt(deg)
  dinv_col = dinv[:N].reshape(N, 1)
  zz16 = jnp.zeros((NPT, D1), jnp.float32)
  zz40 = jnp.zeros((NPT, D2), jnp.float32)

  h1 = _stage_a(x, W1)
  p1 = _pass16(h1, dinv, edges3, zz16)

  b1r = b1.reshape(1, D1)
  b2r = b2.reshape(1, D2)

  hs2 = _stage_b(p1, h1, dinv_col, b1r, W2)
  p2 = _pass40(hs2, edges3, zz40)
  return _stage_c(p2, hs2, dinv_col, b2r).T


# speedup vs baseline: 1.4309x; 1.0228x over previous
import functools

import jax
import jax.numpy as jnp
from jax import lax
from jax.experimental import pallas as pl
from jax.experimental.pallas import tpu as pltpu
from jax.experimental.pallas import tpu_sc as plsc

N = 10000
NDEG = 10240
EROWS = 2500
RPT = 79
NPT = N // 16
D1 = 16
D2 = 40
RBLK = 2000

_MESH = dict(core_axis_name="c", subcore_axis_name="s")
_SC_PARAMS = pltpu.CompilerParams(use_tc_tiling_on_sc=False,
                                  needs_layout_passes=False)


def _tile_rows(wid):
  rb = wid * 78 + jnp.maximum(wid - 28, 0)
  nblk = 78 + (wid >= 28).astype(jnp.int32)
  return rb, nblk


def _make_deg():
  mesh = plsc.VectorSubcoreMesh(**_MESH)

  @functools.partial(
      pl.kernel,
      out_type=jax.ShapeDtypeStruct((2, NDEG), jnp.float32),
      mesh=mesh,
      compiler_params=_SC_PARAMS,
      scratch_types=[
          pltpu.VMEM((RPT, 128), jnp.int32),
          pltpu.VMEM((128,), jnp.float32),
          pltpu.VMEM((NDEG // 16,), jnp.float32),
          pltpu.VMEM_SHARED((NDEG,), jnp.float32),
          pltpu.SemaphoreType.DMA,
      ],
  )
  def deg_kernel(edges_hbm, out_hbm, idx_v, ones_v, zero_v, acc, dsem):
    c = lax.axis_index("c")
    s = lax.axis_index("s")
    wid = c * 16 + s
    rb, nblk = _tile_rows(wid)
    npt = NDEG // 16
    one16 = jnp.full((16,), 1.0, jnp.float32)
    zero16 = jnp.zeros((16,), jnp.float32)
    for i in range(8):
      ones_v[pl.ds(i * 16, 16)] = one16

    def zbody(i, _):
      zero_v[pl.ds(i * 16, 16)] = zero16
      return 0

    lax.fori_loop(0, npt // 16, zbody, 0)
    pltpu.sync_copy(zero_v, acc.at[pl.ds(s * npt, npt)])
    pltpu.sync_copy(edges_hbm.at[pl.ds(rb, RPT), 1], idx_v)
    plsc.subcore_barrier()

    def body(j, _):
      pltpu.async_copy(ones_v, acc.at[idx_v.at[j]], dsem, add=True)
      return 0

    lax.fori_loop(0, nblk, body, 0)

    def drain(j, _):
      pltpu.make_async_copy(ones_v, acc.at[idx_v.at[j]], dsem).wait()
      return 0

    lax.fori_loop(0, nblk, drain, 0)
    plsc.subcore_barrier()
    pltpu.sync_copy(acc.at[pl.ds(s * npt, npt)],
                    out_hbm.at[c, pl.ds(s * npt, npt)])

  return deg_kernel


def _make_pass(d, scaled=False):
  mesh = plsc.VectorSubcoreMesh(**_MESH)
  scratch = [
      pltpu.VMEM((RPT, 128), jnp.int32),
      pltpu.VMEM((RPT, 128), jnp.int32),
      pltpu.VMEM((128, d), jnp.float32),
      pltpu.VMEM((128, d), jnp.float32),
      pltpu.VMEM_SHARED((N, d), jnp.float32),
      pltpu.VMEM_SHARED((N, d), jnp.float32),
      pltpu.SemaphoreType.DMA,
      pltpu.SemaphoreType.DMA,
  ]
  if scaled:
    scratch += [
        pltpu.VMEM((NPT, d), jnp.float32),
        pltpu.VMEM((NPT + 7, ), jnp.float32),
    ]

  def pass_body(hs_hbm, dinv_hbm, edges_hbm, zz_hbm, out_hbm,
                sidx, didx, rows0, rows1, table, acc, sem0, sem1,
                tmp=None, dvec=None):
    c = lax.axis_index("c")
    s = lax.axis_index("s")
    wid = c * 16 + s
    rb, nblk = _tile_rows(wid)
    pltpu.async_copy(zz_hbm, acc.at[pl.ds(s * NPT, NPT)], sem0)
    pltpu.async_copy(edges_hbm.at[pl.ds(rb, RPT), 0], sidx, sem1)
    pltpu.async_copy(edges_hbm.at[pl.ds(rb, RPT), 1], didx, sem1)
    if not scaled:
      pltpu.async_copy(hs_hbm.at[pl.ds(s * NPT, NPT), pl.ds(0, d)],
                       table.at[pl.ds(s * NPT, NPT)], sem0)
      pltpu.make_async_copy(
          hs_hbm.at[pl.ds(s * NPT, NPT), pl.ds(0, d)],
          table.at[pl.ds(s * NPT, NPT)], sem0).wait()
    else:
      pltpu.async_copy(hs_hbm.at[pl.ds(s * NPT, NPT), pl.ds(0, d)], tmp,
                       sem0)
      b8 = s * NPT // 8 * 8
      off = s * NPT - b8
      pltpu.sync_copy(dinv_hbm.at[pl.ds(b8, NPT + 7)], dvec)
      pltpu.make_async_copy(hs_hbm.at[pl.ds(s * NPT, NPT), pl.ds(0, d)],
                            tmp, sem0).wait()

      def scale_row(i, _):
        dv = plsc.load_gather(
            dvec, [jnp.zeros((16,), jnp.int32) + (off + i)])
        tmp[i] = tmp[i] * dv
        return 0

      lax.fori_loop(0, NPT, scale_row, 0)
      pltpu.sync_copy(tmp, table.at[pl.ds(s * NPT, NPT)])
    pltpu.make_async_copy(zz_hbm, acc.at[pl.ds(s * NPT, NPT)], sem0).wait()
    pltpu.make_async_copy(edges_hbm.at[pl.ds(rb, RPT), 0], sidx,
                          sem1).wait()
    pltpu.make_async_copy(edges_hbm.at[pl.ds(rb, RPT), 1], didx,
                          sem1).wait()
    plsc.subcore_barrier()

    pltpu.async_copy(table.at[sidx.at[0]], rows0, sem0)

    def body(i, _):
      b0 = 2 * i
      b1 = 2 * i + 1
      pltpu.async_copy(table.at[sidx.at[b1]], rows1, sem1)
      pltpu.make_async_copy(table.at[sidx.at[b0]], rows0, sem0).wait()
      pltpu.sync_copy(rows0, acc.at[didx.at[b0]], add=True)

      @pl.when(b0 + 2 < nblk)
      def _():
        pltpu.async_copy(table.at[sidx.at[b0 + 2]], rows0, sem0)

      pltpu.make_async_copy(table.at[sidx.at[b1]], rows1, sem1).wait()
      pltpu.sync_copy(rows1, acc.at[didx.at[b1]], add=True)
      return 0

    lax.fori_loop(0, 39, body, 0)

    @pl.when(nblk == RPT)
    def _():
      pltpu.make_async_copy(table.at[sidx.at[RPT - 1]], rows0, sem0).wait()
      pltpu.sync_copy(rows0, acc.at[didx.at[RPT - 1]], add=True)

    plsc.subcore_barrier()
    pltpu.sync_copy(acc.at[pl.ds(s * NPT, NPT)],
                    out_hbm.at[pl.ds(s * NPT, NPT), pl.ds(c * d, d)])

  kern = functools.partial(
      pl.kernel,
      out_type=jax.ShapeDtypeStruct((N, 128), jnp.float32),
      mesh=mesh,
      compiler_params=_SC_PARAMS,
      scratch_types=scratch,
  )
  if scaled:
    return kern(pass_body)

  def body_unscaled(hs_hbm, edges_hbm, zz_hbm, out_hbm, *rest):
    pass_body(hs_hbm, None, edges_hbm, zz_hbm, out_hbm, *rest)

  return kern(body_unscaled)


_deg_call = _make_deg()
_pass16 = _make_pass(D1, scaled=True)
_pass40 = _make_pass(D2)


def _stage_a_body(x_ref, w_ref, o_ref):
  o_ref[:, :D1] = jnp.dot(x_ref[:, :], w_ref[:, :],
                          preferred_element_type=jnp.float32)


def _stage_a(x, w1):
  return pl.pallas_call(
      _stage_a_body,
      out_shape=jax.ShapeDtypeStruct((N, 128), jnp.float32),
  )(x, w1)


def _stage_b_body(p_ref, h_ref, d_ref, b1_ref, w2_ref, o_ref):
  dcol = d_ref[:, :]
  t = ((p_ref[:, :D1] + p_ref[:, D1:2 * D1] + h_ref[:, :D1] * dcol) * dcol
       + b1_ref[:, :])
  t = jnp.maximum(t, 0.0)
  o_ref[:, :D2] = jnp.dot(t, w2_ref[:, :],
                          preferred_element_type=jnp.float32) * dcol


def _stage_b(p1, hs1, dinv_col, b1r, w2):
  return pl.pallas_call(
      _stage_b_body,
      out_shape=jax.ShapeDtypeStruct((N, 128), jnp.float32),
  )(p1, hs1, dinv_col, b1r, w2)


def _stage_c_body(p_ref, h_ref, d_ref, b2_ref, o_ref):
  logits = ((p_ref[:, :D2] + p_ref[:, D2:2 * D2] + h_ref[:, :D2])
            * d_ref[:, :] + b2_ref[:, :])
  m = jnp.max(logits, axis=1, keepdims=True)
  e = jnp.exp(logits - m)
  ssum = jnp.sum(e, axis=1, keepdims=True)
  o_ref[:, :] = (logits - m - jnp.log(ssum)).T


def _stage_c(p2, hs2, dinv_col, b2r):
  return pl.pallas_call(
      _stage_c_body,
      out_shape=jax.ShapeDtypeStruct((D2, N), jnp.float32),
  )(p2, hs2, dinv_col, b2r)


def kernel(x, edge_index, W1, b1, W2, b2):
  edges3 = edge_index.astype(jnp.int32).reshape(2, EROWS, 128).transpose(
      1, 0, 2)

  deg_p = _deg_call(edges3)
  deg = deg_p[0] + deg_p[1] + 1.0
  dinv = lax.rsqrt(deg)
  dinv_col = dinv[:N].reshape(N, 1)
  zz16 = jnp.zeros((NPT, D1), jnp.float32)
  zz40 = jnp.zeros((NPT, D2), jnp.float32)

  h1 = _stage_a(x, W1)
  p1 = _pass16(h1, dinv, edges3, zz16)

  b1r = b1.reshape(1, D1)
  b2r = b2.reshape(1, D2)

  hs2 = _stage_b(p1, h1, dinv_col, b1r, W2)
  p2 = _pass40(hs2, edges3, zz40)
  return _stage_c(p2, hs2, dinv_col, b2r).T


# speedup vs baseline: 1.4333x; 1.0017x over previous
import functools

import jax
import jax.numpy as jnp
from jax import lax
from jax.experimental import pallas as pl
from jax.experimental.pallas import tpu as pltpu
from jax.experimental.pallas import tpu_sc as plsc

N = 10000
NDEG = 10240
EROWS = 2500
RPT = 79
NPT = N // 16
D1 = 16
D2 = 40

_MESH = dict(core_axis_name="c", subcore_axis_name="s")
_SC_PARAMS = pltpu.CompilerParams(use_tc_tiling_on_sc=False,
                                  needs_layout_passes=False)


def _tile_rows(wid):
  rb = wid * 78 + jnp.maximum(wid - 28, 0)
  nblk = 78 + (wid >= 28).astype(jnp.int32)
  return rb, nblk


def _make_deg():
  mesh = plsc.VectorSubcoreMesh(**_MESH)

  @functools.partial(
      pl.kernel,
      out_type=jax.ShapeDtypeStruct((2, NDEG), jnp.float32),
      mesh=mesh,
      compiler_params=_SC_PARAMS,
      scratch_types=[
          pltpu.VMEM((RPT, 128), jnp.int32),
          pltpu.VMEM((128,), jnp.float32),
          pltpu.VMEM((NDEG // 16,), jnp.float32),
          pltpu.VMEM_SHARED((NDEG,), jnp.float32),
          pltpu.SemaphoreType.DMA,
      ],
  )
  def deg_kernel(edges_hbm, out_hbm, idx_v, ones_v, zero_v, acc, dsem):
    c = lax.axis_index("c")
    s = lax.axis_index("s")
    wid = c * 16 + s
    rb, nblk = _tile_rows(wid)
    npt = NDEG // 16
    one16 = jnp.full((16,), 1.0, jnp.float32)
    zero16 = jnp.zeros((16,), jnp.float32)
    for i in range(8):
      ones_v[pl.ds(i * 16, 16)] = one16

    def zbody(i, _):
      zero_v[pl.ds(i * 16, 16)] = zero16
      return 0

    lax.fori_loop(0, npt // 16, zbody, 0)
    pltpu.sync_copy(zero_v, acc.at[pl.ds(s * npt, npt)])
    pltpu.sync_copy(edges_hbm.at[pl.ds(rb, RPT), 1], idx_v)
    plsc.subcore_barrier()

    def body(j, _):
      pltpu.async_copy(ones_v, acc.at[idx_v.at[j]], dsem, add=True)
      return 0

    lax.fori_loop(0, nblk, body, 0)

    def drain(j, _):
      pltpu.make_async_copy(ones_v, acc.at[idx_v.at[j]], dsem).wait()
      return 0

    lax.fori_loop(0, nblk, drain, 0)
    plsc.subcore_barrier()
    pltpu.sync_copy(acc.at[pl.ds(s * npt, npt)],
                    out_hbm.at[c, pl.ds(s * npt, npt)])

  return deg_kernel


def _make_pass(d, scaled=False):
  mesh = plsc.VectorSubcoreMesh(**_MESH)
  scratch = [
      pltpu.VMEM((RPT, 128), jnp.int32),
      pltpu.VMEM((RPT, 128), jnp.int32),
      pltpu.VMEM((128, d), jnp.float32),
      pltpu.VMEM((128, d), jnp.float32),
      pltpu.VMEM_SHARED((N, d), jnp.float32),
      pltpu.VMEM_SHARED((N, d), jnp.float32),
      pltpu.SemaphoreType.DMA,
      pltpu.SemaphoreType.DMA,
  ]
  if scaled:
    scratch += [
        pltpu.VMEM((NPT, d), jnp.float32),
        pltpu.VMEM((NPT + 7, ), jnp.float32),
    ]

  def pass_body(hs_hbm, dinv_hbm, edges_hbm, zz_hbm, out_hbm,
                sidx, didx, rows0, rows1, table, acc, sem0, sem1,
                tmp=None, dvec=None):
    c = lax.axis_index("c")
    s = lax.axis_index("s")
    wid = c * 16 + s
    rb, nblk = _tile_rows(wid)
    pltpu.async_copy(zz_hbm, acc.at[pl.ds(s * NPT, NPT)], sem0)
    pltpu.async_copy(edges_hbm.at[pl.ds(rb, RPT), 0], sidx, sem1)
    pltpu.async_copy(edges_hbm.at[pl.ds(rb, RPT), 1], didx, sem1)
    if not scaled:
      pltpu.async_copy(hs_hbm.at[pl.ds(s * NPT, NPT), pl.ds(0, d)],
                       table.at[pl.ds(s * NPT, NPT)], sem0)
      pltpu.make_async_copy(
          hs_hbm.at[pl.ds(s * NPT, NPT), pl.ds(0, d)],
          table.at[pl.ds(s * NPT, NPT)], sem0).wait()
    else:
      pltpu.async_copy(hs_hbm.at[pl.ds(s * NPT, NPT), pl.ds(0, d)], tmp,
                       sem0)
      b8 = s * NPT // 8 * 8
      off = s * NPT - b8
      pltpu.sync_copy(dinv_hbm.at[pl.ds(b8, NPT + 7)], dvec)
      pltpu.make_async_copy(hs_hbm.at[pl.ds(s * NPT, NPT), pl.ds(0, d)],
                            tmp, sem0).wait()

      def scale_row(i, _):
        dv = plsc.load_gather(
            dvec, [jnp.zeros((16,), jnp.int32) + (off + i)])
        tmp[i] = tmp[i] * dv
        return 0

      lax.fori_loop(0, NPT, scale_row, 0)
      pltpu.sync_copy(tmp, table.at[pl.ds(s * NPT, NPT)])
    pltpu.make_async_copy(zz_hbm, acc.at[pl.ds(s * NPT, NPT)], sem0).wait()
    pltpu.make_async_copy(edges_hbm.at[pl.ds(rb, RPT), 0], sidx,
                          sem1).wait()
    pltpu.make_async_copy(edges_hbm.at[pl.ds(rb, RPT), 1], didx,
                          sem1).wait()
    plsc.subcore_barrier()

    pltpu.async_copy(table.at[sidx.at[0]], rows0, sem0)

    def body(i, _):
      b0 = 2 * i
      b1 = 2 * i + 1
      pltpu.async_copy(table.at[sidx.at[b1]], rows1, sem1)
      pltpu.make_async_copy(table.at[sidx.at[b0]], rows0, sem0).wait()
      pltpu.sync_copy(rows0, acc.at[didx.at[b0]], add=True)

      @pl.when(b0 + 2 < nblk)
      def _():
        pltpu.async_copy(table.at[sidx.at[b0 + 2]], rows0, sem0)

      pltpu.make_async_copy(table.at[sidx.at[b1]], rows1, sem1).wait()
      pltpu.sync_copy(rows1, acc.at[didx.at[b1]], add=True)
      return 0

    lax.fori_loop(0, 39, body, 0)

    @pl.when(nblk == RPT)
    def _():
      pltpu.make_async_copy(table.at[sidx.at[RPT - 1]], rows0, sem0).wait()
      pltpu.sync_copy(rows0, acc.at[didx.at[RPT - 1]], add=True)

    plsc.subcore_barrier()
    pltpu.sync_copy(acc.at[pl.ds(s * NPT, NPT)],
                    out_hbm.at[pl.ds(s * NPT, NPT), pl.ds(c * d, d)])

  kern = functools.partial(
      pl.kernel,
      out_type=jax.ShapeDtypeStruct((N, 128), jnp.float32),
      mesh=mesh,
      compiler_params=_SC_PARAMS,
      scratch_types=scratch,
  )
  if scaled:
    return kern(pass_body)

  def body_unscaled(hs_hbm, edges_hbm, zz_hbm, out_hbm, *rest):
    pass_body(hs_hbm, None, edges_hbm, zz_hbm, out_hbm, *rest)

  return kern(body_unscaled)


_deg_call = _make_deg()
_pass16 = _make_pass(D1, scaled=True)
_pass40 = _make_pass(D2)


def _stage_a_body(x_ref, w_ref, o_ref):
  o_ref[:, :D1] = jnp.dot(x_ref[:, :], w_ref[:, :],
                          preferred_element_type=jnp.float32)


def _stage_a(x, w1):
  return pl.pallas_call(
      _stage_a_body,
      out_shape=jax.ShapeDtypeStruct((N, 128), jnp.float32),
  )(x, w1)


def _stage_b_body(p_ref, h_ref, d_ref, b1_ref, w2_ref, o_ref):
  dcol = d_ref[:, :]
  t = ((p_ref[:, :D1] + p_ref[:, D1:2 * D1] + h_ref[:, :D1] * dcol) * dcol
       + b1_ref[:, :])
  t = jnp.maximum(t, 0.0)
  o_ref[:, :D2] = jnp.dot(t, w2_ref[:, :],
                          preferred_element_type=jnp.float32) * dcol


def _stage_b(p1, hs1, dinv_col, b1r, w2):
  return pl.pallas_call(
      _stage_b_body,
      out_shape=jax.ShapeDtypeStruct((N, 128), jnp.float32),
  )(p1, hs1, dinv_col, b1r, w2)


def _stage_c_body(p_ref, h_ref, d_ref, b2_ref, o_ref):
  logits = ((p_ref[:, :D2] + p_ref[:, D2:2 * D2] + h_ref[:, :D2])
            * d_ref[:, :] + b2_ref[:, :])
  m = jnp.max(logits, axis=1, keepdims=True)
  e = jnp.exp(logits - m)
  ssum = jnp.sum(e, axis=1, keepdims=True)
  o_ref[:, :] = (logits - m - jnp.log(ssum)).T


def _stage_c(p2, hs2, dinv_col, b2r):
  return pl.pallas_call(
      _stage_c_body,
      out_shape=jax.ShapeDtypeStruct((D2, N), jnp.float32),
  )(p2, hs2, dinv_col, b2r)


def kernel(x, edge_index, W1, b1, W2, b2):
  edges3 = edge_index.astype(jnp.int32).reshape(2, EROWS, 128).transpose(
      1, 0, 2)

  deg_p = _deg_call(edges3)
  deg = deg_p[0] + deg_p[1] + 1.0
  dinv = lax.rsqrt(deg)
  dinv_col = dinv[:N].reshape(N, 1)
  zz16 = jnp.zeros((NPT, D1), jnp.float32)
  zz40 = jnp.zeros((NPT, D2), jnp.float32)

  h1 = _stage_a(x, W1)
  p1 = _pass16(h1, dinv, edges3, zz16)

  b1r = b1.reshape(1, D1)
  b2r = b2.reshape(1, D2)

  hs2 = _stage_b(p1, h1, dinv_col, b1r, W2)
  p2 = _pass40(hs2, edges3, zz40)
  return _stage_c(p2, hs2, dinv_col, b2r).T
